# baseline (speedup 1.0000x reference)
import functools

import jax
import jax.numpy as jnp
from jax import lax
from jax.experimental import pallas as pl
from jax.experimental.pallas import tpu as pltpu
from jax.experimental.pallas import tpu_sc as plsc

D = 64
B = 16384
NC, NS, L = 2, 16, 16
NW = NC * NS
BPW = B // NW
CHUNK = 128
NCH = BPW // CHUNK
NG = BPW // L

_mesh = plsc.VectorSubcoreMesh(core_axis_name="c", subcore_axis_name="s")


@functools.partial(
    pl.kernel,
    out_type=jax.ShapeDtypeStruct((NW, L), jnp.float32),
    mesh=_mesh,
    compiler_params=pltpu.CompilerParams(needs_layout_passes=False),
    scratch_types=[
        pltpu.VMEM((NCH, CHUNK), jnp.int32),
        pltpu.VMEM((NCH, CHUNK), jnp.int32),
        pltpu.VMEM((BPW // 2, 2 * D), jnp.float32),
        pltpu.VMEM((BPW, 2 * D), jnp.float32),
        pltpu.VMEM((L,), jnp.float32),
        pltpu.SemaphoreType.DMA,
        pltpu.SemaphoreType.DMA,
    ],
)
def _sc_partials(feat_hbm, lab_hbm, cent_hbm, out_hbm,
                 idx_v, gidx_v, feat_v, rows_v, acc_v, gsem, fsem):
    wid = lax.axis_index("s") * NC + lax.axis_index("c")
    pltpu.sync_copy(lab_hbm.at[pl.ds(wid * NCH, NCH)], idx_v)
    fcp = pltpu.async_copy(
        feat_hbm.at[pl.ds(wid * (BPW // 2), BPW // 2)], feat_v, fsem)
    for k in range(NCH * CHUNK // L):
        r, c0 = k // (CHUNK // L), (k % (CHUNK // L)) * L
        gidx_v[r, pl.ds(c0, L)] = idx_v[r, pl.ds(c0, L)] >> jnp.int32(1)
    gcps = [
        pltpu.async_copy(cent_hbm.at[gidx_v.at[j]],
                         rows_v.at[pl.ds(j * CHUNK, CHUNK)], gsem)
        for j in range(NCH)
    ]
    fcp.wait()
    for g in gcps:
        g.wait()

    lanes = lax.iota(jnp.int32, L)

    def group_body(g, acc):
        p0 = g * L
        pos = p0 + lanes
        lab = plsc.load_gather(
            idx_v, [pos >> jnp.int32(7), pos & jnp.int32(127)])
        gcol0 = (lab & jnp.int32(1)) * jnp.int32(D)
        fcol0 = (pos & jnp.int32(1)) * jnp.int32(D)
        frow = pos >> jnp.int32(1)
        dist = jnp.zeros((L,), jnp.float32)
        for c in range(D):
            f = plsc.load_gather(feat_v, [frow, fcol0 + jnp.int32(c)])
            t = plsc.load_gather(rows_v, [pos, gcol0 + jnp.int32(c)])
            dlt = f - t
            dist = dist + dlt * dlt
        return acc + dist

    acc = lax.fori_loop(0, NG, group_body, jnp.zeros((L,), jnp.float32))
    acc_v[...] = acc
    pltpu.sync_copy(acc_v, out_hbm.at[wid])


def _tc_mean_body(p_ref, o_ref):
    o_ref[0, 0] = jnp.sum(p_ref[...]) * (1.0 / B)


_tc_mean = pl.pallas_call(
    _tc_mean_body,
    out_shape=jax.ShapeDtypeStruct((1, 1), jnp.float32),
    out_specs=pl.BlockSpec(memory_space=pltpu.SMEM),
)


def kernel(features, labels, centers):
    feat2 = features.reshape(B // 2, 2 * D)
    lab2 = labels.astype(jnp.int32).reshape(128, 128)
    cent2 = centers.reshape(-1, 2 * D)
    partials = _sc_partials(feat2, lab2, cent2)
    return _tc_mean(partials)[0, 0]


# speedup vs baseline: 1.2153x; 1.2153x over previous
import functools

import jax
import jax.numpy as jnp
from jax import lax
from jax.experimental import pallas as pl
from jax.experimental.pallas import tpu as pltpu
from jax.experimental.pallas import tpu_sc as plsc

D = 64
B = 16384
NC, NS, L = 2, 16, 16
NW = NC * NS
BPW = B // NW
CHUNK = 128
NCH = BPW // CHUNK
NG = BPW // L

_mesh = plsc.VectorSubcoreMesh(
    core_axis_name="c", subcore_axis_name="s", num_cores=NC, num_subcores=NS)


@functools.partial(
    pl.kernel,
    out_type=jax.ShapeDtypeStruct((NW, L), jnp.float32),
    mesh=_mesh,
    compiler_params=pltpu.CompilerParams(
        needs_layout_passes=False, disable_bounds_checks=True),
    scratch_types=[
        pltpu.VMEM((NCH, CHUNK), jnp.int32),
        pltpu.VMEM((NCH, CHUNK), jnp.int32),
        pltpu.VMEM((D, BPW), jnp.float32),
        pltpu.VMEM((BPW, 2 * D), jnp.float32),
        pltpu.VMEM((L,), jnp.float32),
        pltpu.SemaphoreType.DMA,
        pltpu.SemaphoreType.DMA,
    ],
)
def _sc_partials(featT_hbm, lab_hbm, cent_hbm, out_hbm,
                 idx_v, gidx_v, feat_v, rows_v, acc_v, gsem, fsem):
    wid = lax.axis_index("s") * NC + lax.axis_index("c")
    pltpu.sync_copy(lab_hbm.at[pl.ds(wid * NCH, NCH)], idx_v)
    fcp = pltpu.async_copy(
        featT_hbm.at[:, pl.ds(wid * BPW, BPW)], feat_v, fsem)
    for k in range(NCH * CHUNK // L):
        r, c0 = k // (CHUNK // L), (k % (CHUNK // L)) * L
        gidx_v[r, pl.ds(c0, L)] = idx_v[r, pl.ds(c0, L)] >> jnp.int32(1)
    gcps = [
        pltpu.async_copy(cent_hbm.at[gidx_v.at[j]],
                         rows_v.at[pl.ds(j * CHUNK, CHUNK)], gsem)
        for j in range(NCH)
    ]
    fcp.wait()
    for g in gcps:
        g.wait()

    lanes = lax.iota(jnp.int32, L)

    def group_body(g, acc):
        p0 = g * L
        pos = p0 + lanes
        lab = plsc.load_gather(
            idx_v, [pos >> jnp.int32(7), pos & jnp.int32(127)])
        gcol0 = (lab & jnp.int32(1)) * jnp.int32(D)
        dist = jnp.zeros((L,), jnp.float32)
        for c in range(D):
            f = feat_v[c, pl.ds(p0, L)]
            t = plsc.load_gather(rows_v, [pos, gcol0 + jnp.int32(c)])
            dlt = f - t
            dist = dist + dlt * dlt
        return acc + dist

    acc = lax.fori_loop(0, NG, group_body, jnp.zeros((L,), jnp.float32))
    acc_v[...] = acc
    pltpu.sync_copy(acc_v, out_hbm.at[wid])


def _tc_mean_body(p_ref, o_ref):
    o_ref[0, 0] = jnp.sum(p_ref[...]) * (1.0 / B)


_tc_mean = pl.pallas_call(
    _tc_mean_body,
    out_shape=jax.ShapeDtypeStruct((1, 1), jnp.float32),
    out_specs=pl.BlockSpec(memory_space=pltpu.SMEM),
)


def kernel(features, labels, centers):
    featT = features.T
    lab2 = labels.astype(jnp.int32).reshape(128, 128)
    cent2 = centers.reshape(-1, 2 * D)
    partials = _sc_partials(featT, lab2, cent2)
    return _tc_mean(partials)[0, 0]
